# baseline (speedup 1.0000x reference)
import functools

import jax
import jax.numpy as jnp
from jax import lax
from jax.experimental import pallas as pl
from jax.experimental.pallas import tpu as pltpu
from jax.experimental.pallas import tpu_sc as plsc

B = 16384
F = 64
NC = 2
NS = 16
NW = NC * NS
BPW = B // NW
CH = 128
NCH = BPW // CH
L = 16

N_ROWS = 1000000
TCOLS = 128
HALF = 500096
TGRID = HALF // TCOLS


def _transpose_body(src1_ref, src2_ref, dst_ref):
    x1 = src1_ref[...]
    x2 = src2_ref[...]
    dst_ref[...] = jnp.concatenate(
        [jnp.swapaxes(x1, 0, 1), jnp.swapaxes(x2, 0, 1)], axis=1)


def _tc_transpose(tab_t):
    return pl.pallas_call(
        _transpose_body,
        grid=(TGRID,),
        in_specs=[
            pl.BlockSpec((F, TCOLS), lambda j: (0, j)),
            pl.BlockSpec((F, TCOLS), lambda j: (0, j + TGRID)),
        ],
        out_specs=pl.BlockSpec((TCOLS, TCOLS), lambda j: (j, 0)),
        out_shape=jax.ShapeDtypeStruct((HALF, TCOLS), jnp.float32),
    )(tab_t, tab_t)


def _body(user_hbm, item_hbm, uf_hbm, if_hbm, ub_hbm, ib_hbm, out_hbm,
          uidx, iidx, uhalf, ihalf,
          ubuf, ibuf, ubias, ibias, outv, sem0, sem1):
    c = lax.axis_index("c")
    s = lax.axis_index("s")
    wid = s * NC + c
    row0 = wid * NCH

    pltpu.sync_copy(user_hbm.at[pl.ds(row0, NCH)], uidx)
    pltpu.sync_copy(item_hbm.at[pl.ds(row0, NCH)], iidx)

    for k in range(NCH):
        for m in range(CH // L):
            sl = pl.ds(m * L, L)
            uv = uidx[k, sl]
            uhalf[k, sl] = jnp.where(uv >= HALF, uv - HALF, uv)
            ihalf[k, sl] = jax.lax.shift_right_logical(iidx[k, sl], 1)

    sems = (sem0, sem1)

    def fire(k):
        bsel = k % 2
        sem = sems[bsel]
        cps = (
            pltpu.async_copy(uf_hbm.at[uhalf.at[k]], ubuf.at[bsel], sem),
            pltpu.async_copy(if_hbm.at[ihalf.at[k]], ibuf.at[bsel], sem),
            pltpu.async_copy(ub_hbm.at[uidx.at[k]], ubias.at[k], sem),
            pltpu.async_copy(ib_hbm.at[iidx.at[k]], ibias.at[k], sem),
        )
        return cps

    lane = lax.iota(jnp.int32, L)
    inflight = fire(0)

    for k in range(NCH):
        for cp in inflight:
            cp.wait()
        if k + 1 < NCH:
            nxt = fire(k + 1)
        bsel = k % 2
        ub_k = ubuf.at[bsel]
        ib_k = ibuf.at[bsel]

        def grp(g, carry, ub_k=ub_k, ib_k=ib_k, k=k):
            base = g * L
            sl = pl.ds(base, L)
            rows = base + lane
            offu = jnp.where(uidx[k, sl] >= HALF, F, 0)
            offi = jax.lax.shift_left(iidx[k, sl] & 1, 6)
            acc = None
            for f in range(F):
                cu = plsc.load_gather(ub_k, [rows, offu + f])
                ci = plsc.load_gather(ib_k, [rows, offi + f])
                acc = cu * ci if acc is None else acc + cu * ci
            outv[k, sl] = acc + ubias[k, sl] + ibias[k, sl]
            return carry

        lax.fori_loop(0, CH // L, grp, None)
        if k + 1 < NCH:
            inflight = nxt

    pltpu.sync_copy(outv, out_hbm.at[pl.ds(row0, NCH)])


@jax.jit
def _sc_call(user2, item2, uf2, if2, ub, ib):
    grid_kernel = functools.partial(
        pl.kernel,
        out_type=jax.ShapeDtypeStruct((B // CH, CH), jnp.float32),
        mesh=plsc.VectorSubcoreMesh(core_axis_name="c", subcore_axis_name="s"),
        compiler_params=pltpu.CompilerParams(
            needs_layout_passes=False, use_tc_tiling_on_sc=True),
        scratch_types=[
            pltpu.VMEM((NCH, CH), jnp.int32),
            pltpu.VMEM((NCH, CH), jnp.int32),
            pltpu.VMEM((NCH, CH), jnp.int32),
            pltpu.VMEM((NCH, CH), jnp.int32),
            pltpu.VMEM((2, CH, TCOLS), jnp.float32),
            pltpu.VMEM((2, CH, TCOLS), jnp.float32),
            pltpu.VMEM((NCH, CH), jnp.float32),
            pltpu.VMEM((NCH, CH), jnp.float32),
            pltpu.VMEM((NCH, CH), jnp.float32),
            pltpu.SemaphoreType.DMA,
            pltpu.SemaphoreType.DMA,
        ],
    )
    return grid_kernel(_body)(user2, item2, uf2, if2, ub, ib)


def kernel(user, item, user_factors, item_factors, users_biases, items_biases):
    user2 = user.astype(jnp.int32).reshape(B // CH, CH)
    item2 = item.astype(jnp.int32).reshape(B // CH, CH)
    uf2 = _tc_transpose(user_factors.T)
    if2 = item_factors.reshape(N_ROWS // 2, TCOLS)
    ub = users_biases.reshape(-1)
    ib = items_biases.reshape(-1)
    out = _sc_call(user2, item2, uf2, if2, ub, ib)
    return out.reshape(-1)


# speedup vs baseline: 2.4514x; 2.4514x over previous
import functools

import jax
import jax.numpy as jnp
from jax import lax
from jax.experimental import pallas as pl
from jax.experimental.pallas import tpu as pltpu
from jax.experimental.pallas import tpu_sc as plsc

B = 16384
F = 64
NC = 2
NS = 16
NW = NC * NS
BPW = B // NW
CH = 128
NCH = BPW // CH
L = 16

N_ROWS = 1000000
TCOLS = 128
HALF = 500096
TGRID = HALF // TCOLS


def _transpose_body(src1_ref, src2_ref, dst_ref):
    x1 = src1_ref[...]
    x2 = src2_ref[...]
    dst_ref[...] = jnp.concatenate(
        [jnp.swapaxes(x1, 0, 1), jnp.swapaxes(x2, 0, 1)], axis=1)


def _tc_transpose(tab_t):
    return pl.pallas_call(
        _transpose_body,
        grid=(TGRID,),
        in_specs=[
            pl.BlockSpec((F, TCOLS), lambda j: (0, j)),
            pl.BlockSpec((F, TCOLS), lambda j: (0, j + TGRID)),
        ],
        out_specs=pl.BlockSpec((TCOLS, TCOLS), lambda j: (j, 0)),
        out_shape=jax.ShapeDtypeStruct((HALF, TCOLS), jnp.float32),
    )(tab_t, tab_t)


def _body(user_hbm, item_hbm, uf_hbm, if_hbm, ub_hbm, ib_hbm, out_hbm,
          uidx, iidx, uhalf, ihalf,
          ubuf, ibuf, ubias, ibias, outv, sem0, sem1):
    c = lax.axis_index("c")
    s = lax.axis_index("s")
    wid = s * NC + c
    row0 = wid * NCH

    pltpu.sync_copy(user_hbm.at[pl.ds(row0, NCH)], uidx)
    pltpu.sync_copy(item_hbm.at[pl.ds(row0, NCH)], iidx)

    for k in range(NCH):
        for m in range(CH // L):
            sl = pl.ds(m * L, L)
            uhalf[k, sl] = jax.lax.shift_right_logical(uidx[k, sl], 1)
            ihalf[k, sl] = jax.lax.shift_right_logical(iidx[k, sl], 1)

    sems = (sem0, sem1)

    def fire(k):
        bsel = k % 2
        sem = sems[bsel]
        cps = (
            pltpu.async_copy(uf_hbm.at[uhalf.at[k]], ubuf.at[bsel], sem),
            pltpu.async_copy(if_hbm.at[ihalf.at[k]], ibuf.at[bsel], sem),
            pltpu.async_copy(ub_hbm.at[uidx.at[k]], ubias.at[k], sem),
            pltpu.async_copy(ib_hbm.at[iidx.at[k]], ibias.at[k], sem),
        )
        return cps

    lane = lax.iota(jnp.int32, L)
    inflight = fire(0)

    for k in range(NCH):
        for cp in inflight:
            cp.wait()
        if k + 1 < NCH:
            nxt = fire(k + 1)
        bsel = k % 2
        ub_k = ubuf.at[bsel]
        ib_k = ibuf.at[bsel]

        def grp(g, carry, ub_k=ub_k, ib_k=ib_k, k=k):
            base = g * L
            sl = pl.ds(base, L)
            rows = base + lane
            offu = jax.lax.shift_left(uidx[k, sl] & 1, 6)
            offi = jax.lax.shift_left(iidx[k, sl] & 1, 6)
            acc = None
            for f in range(F):
                cu = plsc.load_gather(ub_k, [rows, offu + f])
                ci = plsc.load_gather(ib_k, [rows, offi + f])
                acc = cu * ci if acc is None else acc + cu * ci
            outv[k, sl] = acc + ubias[k, sl] + ibias[k, sl]
            return carry

        lax.fori_loop(0, CH // L, grp, None)
        if k + 1 < NCH:
            inflight = nxt

    pltpu.sync_copy(outv, out_hbm.at[pl.ds(row0, NCH)])


@jax.jit
def _sc_call(user2, item2, uf2, if2, ub, ib):
    grid_kernel = functools.partial(
        pl.kernel,
        out_type=jax.ShapeDtypeStruct((B // CH, CH), jnp.float32),
        mesh=plsc.VectorSubcoreMesh(core_axis_name="c", subcore_axis_name="s"),
        compiler_params=pltpu.CompilerParams(
            needs_layout_passes=False, use_tc_tiling_on_sc=True),
        scratch_types=[
            pltpu.VMEM((NCH, CH), jnp.int32),
            pltpu.VMEM((NCH, CH), jnp.int32),
            pltpu.VMEM((NCH, CH), jnp.int32),
            pltpu.VMEM((NCH, CH), jnp.int32),
            pltpu.VMEM((2, CH, TCOLS), jnp.float32),
            pltpu.VMEM((2, CH, TCOLS), jnp.float32),
            pltpu.VMEM((NCH, CH), jnp.float32),
            pltpu.VMEM((NCH, CH), jnp.float32),
            pltpu.VMEM((NCH, CH), jnp.float32),
            pltpu.SemaphoreType.DMA,
            pltpu.SemaphoreType.DMA,
        ],
    )
    return grid_kernel(_body)(user2, item2, uf2, if2, ub, ib)


def kernel(user, item, user_factors, item_factors, users_biases, items_biases):
    user2 = user.astype(jnp.int32).reshape(B // CH, CH)
    item2 = item.astype(jnp.int32).reshape(B // CH, CH)
    uf2 = user_factors.reshape(N_ROWS // 2, TCOLS)
    if2 = item_factors.reshape(N_ROWS // 2, TCOLS)
    ub = users_biases.reshape(-1)
    ib = items_biases.reshape(-1)
    out = _sc_call(user2, item2, uf2, if2, ub, ib)
    return out.reshape(-1)
